# baseline (speedup 1.0000x reference)
import functools

import jax
import jax.numpy as jnp
from jax import lax
from jax.experimental import pallas as pl
from jax.experimental.pallas import tpu as pltpu
from jax.experimental.pallas import tpu_sc as plsc

_RX = _RY = _RZ = 72
_N, _C, _H, _W = 16, 3, 224, 224
_HW = _H * _W
_V = _RX * _RY * _RZ
_NW = 32
_PW = _N * _HW // _NW
_P = 128
_NCHUNK = _PW // _P
_G = _P // 16

_OFFS = (0, 1, _RZ, _RZ + 1, _RY * _RZ, _RY * _RZ + 1,
         _RY * _RZ + _RZ, _RY * _RZ + _RZ + 1)

_mesh = plsc.VectorSubcoreMesh(core_axis_name="c", subcore_axis_name="s")


@functools.partial(
    pl.kernel,
    out_type=jax.ShapeDtypeStruct((_N * _C, _HW), jnp.float32),
    mesh=_mesh,
    scratch_types=[
        pltpu.VMEM((_C, _P), jnp.float32),
        pltpu.VMEM((8, _P), jnp.int32),
        pltpu.VMEM((_C, _P), jnp.float32),
        [pltpu.VMEM((_P, 8), jnp.float32) for _ in range(8)],
        pltpu.VMEM((_C, _P), jnp.float32),
        pltpu.SemaphoreType.DMA,
    ],
    compiler_params=pltpu.CompilerParams(use_tc_tiling_on_sc=False,
                                         needs_layout_passes=False),
)
def _lut_kernel(imgs_hbm, lut_hbm, out_hbm, inbuf, idxbuf, fbuf, gbuf, obuf,
                sem):
    wid = lax.axis_index("s") * 2 + lax.axis_index("c")
    img = wid // 2
    base0 = (wid % 2) * _PW
    lut_base = img * _V
    iota = lax.iota(jnp.int32, 16)

    @pl.loop(0, _NCHUNK)
    def _chunk(ci):
        base = base0 + ci * _P
        for c in range(_C):
            pltpu.sync_copy(imgs_hbm.at[img * _C + c, pl.ds(base, _P)],
                            inbuf.at[c])

        for g in range(_G):
            sl = pl.ds(g * 16, 16)
            sr = inbuf[0, sl] * float(_RX - 1)
            sg = inbuf[1, sl] * float(_RY - 1)
            sb = inbuf[2, sl] * float(_RZ - 1)
            ir = jnp.minimum(sr.astype(jnp.int32), _RX - 2)
            ig = jnp.minimum(sg.astype(jnp.int32), _RY - 2)
            ib = jnp.minimum(sb.astype(jnp.int32), _RZ - 2)
            fbuf[0, sl] = sr - ir.astype(jnp.float32)
            fbuf[1, sl] = sg - ig.astype(jnp.float32)
            fbuf[2, sl] = sb - ib.astype(jnp.float32)
            idx0 = (ir * _RY + ig) * _RZ + ib + lut_base
            for k, off in enumerate(_OFFS):
                idxbuf[k, sl] = idx0 + off

        copies = [pltpu.async_copy(lut_hbm.at[idxbuf.at[k]], gbuf[k], sem)
                  for k in range(8)]
        for cp in copies:
            cp.wait()

        for g in range(_G):
            sl = pl.ds(g * 16, 16)
            fx = fbuf[0, sl]
            fy = fbuf[1, sl]
            fz = fbuf[2, sl]
            wy0z0 = (1.0 - fy) * (1.0 - fz)
            wy0z1 = (1.0 - fy) * fz
            wy1z0 = fy * (1.0 - fz)
            wy1z1 = fy * fz
            w = [(1.0 - fx) * wy0z0, (1.0 - fx) * wy0z1,
                 (1.0 - fx) * wy1z0, (1.0 - fx) * wy1z1,
                 fx * wy0z0, fx * wy0z1, fx * wy1z0, fx * wy1z1]
            pix = iota + g * 16
            for c in range(_C):
                csplat = jnp.full((16,), c, jnp.int32)
                acc = jnp.zeros((16,), jnp.float32)
                for k in range(8):
                    v = plsc.load_gather(gbuf[k], [pix, csplat])
                    acc = acc + w[k] * v
                obuf[c, sl] = jnp.clip(acc, 0.0, 1.0)

        for c in range(_C):
            pltpu.sync_copy(obuf.at[c],
                            out_hbm.at[img * _C + c, pl.ds(base, _P)])


def kernel(imgs, xform_params):
    imgs_f = imgs.reshape(_N * _C, _HW)
    lut = jnp.pad(xform_params.reshape(_N * _V, _C), ((0, 0), (0, 5)))
    out = _lut_kernel(imgs_f, lut)
    return out.reshape(_N, _C, _H, _W)
